# baseline (speedup 1.0000x reference)
import functools

import jax
import jax.numpy as jnp
from jax import lax
from jax.experimental import pallas as pl
from jax.experimental.pallas import tpu as pltpu
from jax.experimental.pallas import tpu_sc as plsc

_R = 100000
_C = 100
_B = 16384
_DIM = 128

_info = plsc.get_sparse_core_info()
_NC, _NS, _L = _info.num_cores, _info.num_subcores, _info.num_lanes
_NW = _NC * _NS
_BPW = _B // _NW
_CH = 32
_NCHUNK = _BPW // _CH

_sc_mesh = plsc.VectorSubcoreMesh(core_axis_name="c", subcore_axis_name="s")


@functools.partial(
    pl.kernel,
    out_type=jax.ShapeDtypeStruct((8, _B), jnp.float32),
    mesh=_sc_mesh,
    compiler_params=pltpu.CompilerParams(needs_layout_passes=False),
    scratch_types=[
        pltpu.VMEM((_BPW,), jnp.int32),
        pltpu.VMEM((_BPW,), jnp.int32),
        pltpu.VMEM((_CH, 8, _C), jnp.float32),
        pltpu.VMEM((_CH, 8, _C), jnp.float32),
        pltpu.VMEM((8, _BPW), jnp.float32),
        pltpu.SemaphoreType.DMA,
        pltpu.SemaphoreType.DMA,
    ],
)
def _sc_gather(r_hbm, c_hbm, t0, t1, t2, t3, t4, dummy_hbm, feat_hbm,
               ri, ci, stga, stgb, gall,
               sem_a, sem_b):
    wid = lax.axis_index("s") * _NC + lax.axis_index("c")
    base = wid * _BPW
    pltpu.sync_copy(r_hbm.at[pl.ds(base, _BPW)], ri)
    pltpu.sync_copy(c_hbm.at[pl.ds(base, _BPW)], ci)

    zero16 = jnp.zeros((_L,), jnp.float32)
    for rr in range(5, 8):
        for i in range(_BPW // _L):
            gall[rr, pl.ds(i * _L, _L)] = zero16

    tabs = (t0, t1, t2, t3, t4)
    stages = (stga, stgb)
    sems = (sem_a, sem_b)

    def fire(tab, j, buf):
        stg, sem = stages[buf], sems[buf]
        for gg in range(_CH // _L):
            bands = lax.shift_right_logical(
                ri[pl.ds(j * _CH + gg * _L, _L)], 3)
            for k in range(_L):
                pltpu.async_copy(
                    tab.at[pl.ds(bands[k], 1), :, :],
                    stg.at[pl.ds(gg * _L + k, 1), :, :],
                    sem)

    def pick(t, j, buf):
        stg = stages[buf]
        pltpu.make_async_copy(dummy_hbm, stg, sems[buf]).wait()
        for gg in range(_CH // _L):
            sl = pl.ds(j * _CH + gg * _L, _L)
            slots = lax.iota(jnp.int32, _L) + gg * _L
            rows = lax.bitwise_and(ri[sl], 7)
            gall[t, sl] = plsc.load_gather(stg, [slots, rows, ci[sl]])

    for t in range(5):
        tab = tabs[t]
        fire(tab, 0, 0)

        def body(p, carry, tab=tab, t=t):
            fire(tab, 2 * p + 1, 1)
            pick(t, 2 * p, 0)
            fire(tab, 2 * p + 2, 0)
            pick(t, 2 * p + 1, 1)
            return carry

        lax.fori_loop(0, _NCHUNK // 2 - 1, body, 0)
        fire(tab, _NCHUNK - 1, 1)
        pick(t, _NCHUNK - 2, 0)
        pick(t, _NCHUNK - 1, 1)

    pltpu.sync_copy(gall, feat_hbm.at[:, pl.ds(base, _BPW)])


_BLK = 2048


def _fuse_body(ft_ref, wt_ref, b_ref, out_ref):
    out_ref[...] = lax.dot_general(
        ft_ref[...], wt_ref[...],
        (((0,), (0,)), ((), ())),
        preferred_element_type=jnp.float32,
    ) + b_ref[...]


def _fuse(feat, wt, b2):
    return pl.pallas_call(
        _fuse_body,
        out_shape=jax.ShapeDtypeStruct((_B, _DIM), jnp.float32),
        grid=(_B // _BLK,),
        in_specs=[
            pl.BlockSpec((8, _BLK), lambda i: (0, i)),
            pl.BlockSpec((8, _DIM), lambda i: (0, 0)),
            pl.BlockSpec((1, _DIM), lambda i: (0, 0)),
        ],
        out_specs=pl.BlockSpec((_BLK, _DIM), lambda i: (i, 0)),
    )(feat, wt, b2)


def kernel(r_idx, c_idx, cf, ff, fr, sim, cc, W, b):
    r32 = r_idx.astype(jnp.int32)
    c32 = c_idx.astype(jnp.int32)
    dummy = jnp.zeros((_CH, 8, _C), jnp.float32)
    tabs = [t.reshape(_R // 8, 8, _C) for t in (sim, cc, cf, ff, fr)]
    feat8 = _sc_gather(r32, c32, *tabs, dummy)
    wt8 = jnp.pad(W.T, ((0, 3), (0, 0)))
    return _fuse(feat8, wt8, b.reshape(1, _DIM))


# speedup vs baseline: 3.2116x; 3.2116x over previous
import functools

import jax
import jax.numpy as jnp
from jax import lax
from jax.experimental import pallas as pl
from jax.experimental.pallas import tpu as pltpu
from jax.experimental.pallas import tpu_sc as plsc

_R = 100000
_C = 100
_B = 16384
_DIM = 128

_info = plsc.get_sparse_core_info()
_NC, _NS, _L = _info.num_cores, _info.num_subcores, _info.num_lanes
_NW = _NC * _NS
_BPW = _B // _NW
_CH = 32
_NCHUNK = _BPW // _CH

_sc_mesh = plsc.VectorSubcoreMesh(core_axis_name="c", subcore_axis_name="s")


@functools.partial(
    pl.kernel,
    out_type=jax.ShapeDtypeStruct((_B,), jnp.float32),
    mesh=_sc_mesh,
    compiler_params=pltpu.CompilerParams(needs_layout_passes=False),
    scratch_types=[
        pltpu.VMEM((_BPW,), jnp.int32),
        pltpu.VMEM((_BPW,), jnp.int32),
        pltpu.VMEM((_CH * 8, _C), jnp.float32),
        pltpu.VMEM((_CH * 8, _C), jnp.float32),
        pltpu.VMEM((_BPW,), jnp.float32),
        pltpu.SemaphoreType.DMA,
        pltpu.SemaphoreType.DMA,
    ],
)
def _sc_gather1(r_hbm, c_hbm, tab, dummy_hbm, out_hbm,
                ri, ci, stga, stgb, g, sem_a, sem_b):
    wid = lax.axis_index("s") * _NC + lax.axis_index("c")
    base = wid * _BPW
    pltpu.sync_copy(r_hbm.at[pl.ds(base, _BPW)], ri)
    pltpu.sync_copy(c_hbm.at[pl.ds(base, _BPW)], ci)

    stages = (stga, stgb)
    sems = (sem_a, sem_b)

    def fire(j, buf):
        stg, sem = stages[buf], sems[buf]
        for gg in range(_CH // _L):
            bands = lax.shift_left(
                lax.shift_right_logical(
                    ri[pl.ds(j * _CH + gg * _L, _L)], 3), 3)
            for k in range(_L):
                band = pl.multiple_of(bands[k], 8)
                pltpu.async_copy(
                    tab.at[pl.ds(band, 8), :],
                    stg.at[pl.ds((gg * _L + k) * 8, 8), :],
                    sem)

    def pick(j, buf):
        stg = stages[buf]
        pltpu.make_async_copy(dummy_hbm, stg, sems[buf]).wait()
        for gg in range(_CH // _L):
            sl = pl.ds(j * _CH + gg * _L, _L)
            rows = lax.shift_left(lax.iota(jnp.int32, _L) + gg * _L, 3) + \
                lax.bitwise_and(ri[sl], 7)
            g[sl] = plsc.load_gather(stg, [rows, ci[sl]])

    fire(0, 0)

    def body(p, carry):
        fire(2 * p + 1, 1)
        pick(2 * p, 0)
        fire(2 * p + 2, 0)
        pick(2 * p + 1, 1)
        return carry

    lax.fori_loop(0, _NCHUNK // 2 - 1, body, 0)
    fire(_NCHUNK - 1, 1)
    pick(_NCHUNK - 2, 0)
    pick(_NCHUNK - 1, 1)

    pltpu.sync_copy(g, out_hbm.at[pl.ds(base, _BPW)])


_BLK = 2048


def _fuse_body(ft_ref, wt_ref, b_ref, out_ref):
    out_ref[...] = lax.dot_general(
        ft_ref[...], wt_ref[...],
        (((0,), (0,)), ((), ())),
        preferred_element_type=jnp.float32,
    ) + b_ref[...]


def _fuse(feat, wt, b2):
    return pl.pallas_call(
        _fuse_body,
        out_shape=jax.ShapeDtypeStruct((_B, _DIM), jnp.float32),
        grid=(_B // _BLK,),
        in_specs=[
            pl.BlockSpec((5, _BLK), lambda i: (0, i)),
            pl.BlockSpec((5, _DIM), lambda i: (0, 0)),
            pl.BlockSpec((1, _DIM), lambda i: (0, 0)),
        ],
        out_specs=pl.BlockSpec((_BLK, _DIM), lambda i: (i, 0)),
    )(feat, wt, b2)


def kernel(r_idx, c_idx, cf, ff, fr, sim, cc, W, b):
    r32 = r_idx.astype(jnp.int32)
    c32 = c_idx.astype(jnp.int32)
    dummy = jnp.zeros((_CH * 8, _C), jnp.float32)
    feats = [_sc_gather1(r32, c32, t, dummy) for t in (sim, cc, cf, ff, fr)]
    feat = jnp.stack(feats)
    return _fuse(feat, W.T, b.reshape(1, _DIM))
